# baseline (speedup 1.0000x reference)
import functools

import jax
import jax.numpy as jnp
from jax import lax
from jax.experimental import pallas as pl
from jax.experimental.pallas import tpu as pltpu
from jax.experimental.pallas import tpu_sc as plsc

_NUM_EXPERTS = 64
_TOKENS = 8192
_TOP_P = 0.8
_AUX_LOSS_COEF = 0.01

_NC = 2
_NS = 16
_L = 16
_NW = _NC * _NS
_TPW = _TOKENS // _NW
_KV = _NUM_EXPERTS // _L


def _merge16(a, b):
  rb = lax.rev(b, (0,))
  lo = jnp.minimum(a, rb)
  hi = jnp.maximum(a, rb)
  return lax.sort(lo), lax.sort(hi)


def _merge32(a0, a1, b0, b1):
  rb0 = lax.rev(b1, (0,))
  rb1 = lax.rev(b0, (0,))
  lo0 = jnp.minimum(a0, rb0)
  lo1 = jnp.minimum(a1, rb1)
  hi0 = jnp.maximum(a0, rb0)
  hi1 = jnp.maximum(a1, rb1)
  m0 = jnp.minimum(lo0, lo1)
  m1 = jnp.maximum(lo0, lo1)
  m2 = jnp.minimum(hi0, hi1)
  m3 = jnp.maximum(hi0, hi1)
  return lax.sort(m0), lax.sort(m1), lax.sort(m2), lax.sort(m3)


def _sort64(v):
  s = [lax.sort(x) for x in v]
  a0, a1 = _merge16(s[0], s[1])
  b0, b1 = _merge16(s[2], s[3])
  return _merge32(a0, a1, b0, b1)


def _token_step(lg_v, t, accs):
  off = t * _NUM_EXPERTS
  u = [jnp.exp(lg_v[pl.ds(off + k * _L, _L)]) for k in range(_KV)]
  s_tot = jnp.sum(u[0] + u[1] + u[2] + u[3])
  r = _sort64(u)
  t0 = jnp.sum(r[0])
  t1 = jnp.sum(r[1])
  t2 = jnp.sum(r[2])
  c0 = plsc.cumsum(r[0])
  c1 = plsc.cumsum(r[1]) + t0
  c2 = plsc.cumsum(r[2]) + (t0 + t1)
  c3 = plsc.cumsum(r[3]) + (t0 + t1 + t2)
  thr = (1.0 - _TOP_P) * s_tot
  big = jnp.full((_L,), 3.0e38, jnp.float32)
  cand = jnp.minimum(
      jnp.minimum(jnp.where(c0 >= thr, r[0], big),
                  jnp.where(c1 >= thr, r[1], big)),
      jnp.minimum(jnp.where(c2 >= thr, r[2], big),
                  jnp.where(c3 >= thr, r[3], big)))
  tau = jnp.min(cand)
  rinv = jnp.ones((_L,), jnp.float32) / s_tot
  new = []
  for k in range(_KV):
    cnt, sw = accs[2 * k], accs[2 * k + 1]
    new.append(cnt + jnp.where(u[k] >= tau, 1.0, 0.0))
    new.append(sw + u[k] * rinv)
  return tuple(new)


def _two_token_step(lg_v, i, accs):
  mid = _token_step(lg_v, 2 * i, accs)
  return _token_step(lg_v, 2 * i + 1, mid)


def _sc_body(lg_hbm, out_hbm, lg_v, st_v):
  wid = lax.axis_index("s") * _NC + lax.axis_index("c")
  base = wid * _TPW * _NUM_EXPERTS
  pltpu.sync_copy(lg_hbm.at[pl.ds(base, _TPW * _NUM_EXPERTS)], lg_v)

  zero = jnp.zeros((_L,), jnp.float32)
  init = tuple(zero for _ in range(2 * _KV))
  accs = lax.fori_loop(0, _TPW // 2,
                       functools.partial(_two_token_step, lg_v), init)

  for k in range(_KV):
    st_v[k, :] = accs[2 * k]
    st_v[_KV + k, :] = accs[2 * k + 1]
  pltpu.sync_copy(st_v, out_hbm.at[wid])


def _sc_partials(gate_logits_flat):
  mesh = plsc.VectorSubcoreMesh(
      core_axis_name="c", subcore_axis_name="s",
      num_cores=_NC, num_subcores=_NS)
  kfn = pl.kernel(
      _sc_body,
      out_type=jax.ShapeDtypeStruct((_NW, 2 * _KV, _L), jnp.float32),
      mesh=mesh,
      scratch_types=[
          pltpu.VMEM((_TPW * _NUM_EXPERTS,), jnp.float32),
          pltpu.VMEM((2 * _KV, _L), jnp.float32),
      ],
      compiler_params=pltpu.CompilerParams(needs_layout_passes=False),
  )
  return kfn(gate_logits_flat)


def _finish_body(p_ref, o_ref):
  p = p_ref[...]
  cnt = jnp.sum(p[:, :_KV, :], axis=0)
  sw = jnp.sum(p[:, _KV:, :], axis=0)
  scale = _AUX_LOSS_COEF * _NUM_EXPERTS / float(_TOKENS) / float(_TOKENS)
  o_ref[...] = (jnp.sum(cnt * sw) * scale).reshape(1, 1)


def _finish(partials):
  return pl.pallas_call(
      _finish_body,
      out_shape=jax.ShapeDtypeStruct((1, 1), jnp.float32),
  )(partials)


def _burn_body(x_ref, o_ref):
  @pl.when(pl.program_id(0) == 0)
  def _zero():
    o_ref[...] = jnp.zeros((1, 1), jnp.float32)
  x = x_ref[...]
  acc = jnp.exp(x)
  for c in (0.9, 0.8, 0.7, 0.6, 0.5):
    acc = acc + jnp.exp(x * c)
  o_ref[...] += jnp.sum(acc).reshape(1, 1)


def _burn(gate_logits):
  return pl.pallas_call(
      _burn_body,
      grid=(16,),
      in_specs=[pl.BlockSpec((512, 64), lambda i: (i, 0))],
      out_specs=pl.BlockSpec((1, 1), lambda i: (0, 0)),
      out_shape=jax.ShapeDtypeStruct((1, 1), jnp.float32),
  )(gate_logits)


@jax.jit
def kernel(gate_logits, attention_mask):
  del attention_mask
  partials = _sc_partials(gate_logits.reshape(-1))
  b = _burn(gate_logits)
  return _finish(partials)[0, 0] + 0.0 * b[0, 0]


# speedup vs baseline: 1.2010x; 1.2010x over previous
import functools

import jax
import jax.numpy as jnp
from jax import lax
from jax.experimental import pallas as pl
from jax.experimental.pallas import tpu as pltpu
from jax.experimental.pallas import tpu_sc as plsc

_NUM_EXPERTS = 64
_TOKENS = 8192
_TOP_P = 0.8
_AUX_LOSS_COEF = 0.01

_NC = 2
_NS = 16
_L = 16
_NW = _NC * _NS
_TPW = _TOKENS // _NW
_KV = _NUM_EXPERTS // _L


def _merge16(a, b):
  rb = lax.rev(b, (0,))
  lo = jnp.minimum(a, rb)
  hi = jnp.maximum(a, rb)
  return lax.sort(lo), lax.sort(hi)


def _merge32(a0, a1, b0, b1):
  rb0 = lax.rev(b1, (0,))
  rb1 = lax.rev(b0, (0,))
  lo0 = jnp.minimum(a0, rb0)
  lo1 = jnp.minimum(a1, rb1)
  hi0 = jnp.maximum(a0, rb0)
  hi1 = jnp.maximum(a1, rb1)
  m0 = jnp.minimum(lo0, lo1)
  m1 = jnp.maximum(lo0, lo1)
  m2 = jnp.minimum(hi0, hi1)
  m3 = jnp.maximum(hi0, hi1)
  return lax.sort(m0), lax.sort(m1), lax.sort(m2), lax.sort(m3)


def _sort64(v):
  s = [lax.sort(x) for x in v]
  a0, a1 = _merge16(s[0], s[1])
  b0, b1 = _merge16(s[2], s[3])
  return _merge32(a0, a1, b0, b1)


def _token_step(lg_v, t, accs):
  off = t * _NUM_EXPERTS
  u = [jnp.exp(lg_v[pl.ds(off + k * _L, _L)]) for k in range(_KV)]
  s_tot = jnp.sum(u[0] + u[1] + u[2] + u[3])
  r = _sort64(u)
  t0 = jnp.sum(r[0])
  t1 = jnp.sum(r[1])
  t2 = jnp.sum(r[2])
  c0 = plsc.cumsum(r[0])
  c1 = plsc.cumsum(r[1]) + t0
  c2 = plsc.cumsum(r[2]) + (t0 + t1)
  c3 = plsc.cumsum(r[3]) + (t0 + t1 + t2)
  thr = (1.0 - _TOP_P) * s_tot
  big = jnp.full((_L,), 3.0e38, jnp.float32)
  cand = jnp.minimum(
      jnp.minimum(jnp.where(c0 >= thr, r[0], big),
                  jnp.where(c1 >= thr, r[1], big)),
      jnp.minimum(jnp.where(c2 >= thr, r[2], big),
                  jnp.where(c3 >= thr, r[3], big)))
  tau = jnp.min(cand)
  rinv = jnp.ones((_L,), jnp.float32) / s_tot
  new = []
  for k in range(_KV):
    cnt, sw = accs[2 * k], accs[2 * k + 1]
    new.append(cnt + jnp.where(u[k] >= tau, 1.0, 0.0))
    new.append(sw + u[k] * rinv)
  return tuple(new)


def _two_token_step(lg_v, i, accs):
  mid = _token_step(lg_v, 2 * i, accs)
  return _token_step(lg_v, 2 * i + 1, mid)


def _sc_body(lg_hbm, out_hbm, lg_v, st_v):
  wid = lax.axis_index("s") * _NC + lax.axis_index("c")
  base = wid * _TPW * _NUM_EXPERTS
  pltpu.sync_copy(lg_hbm.at[pl.ds(base, _TPW * _NUM_EXPERTS)], lg_v)

  zero = jnp.zeros((_L,), jnp.float32)
  init = tuple(zero for _ in range(2 * _KV))
  accs = lax.fori_loop(0, _TPW // 2,
                       functools.partial(_two_token_step, lg_v), init)

  for k in range(_KV):
    st_v[k, :] = accs[2 * k]
    st_v[_KV + k, :] = accs[2 * k + 1]
  pltpu.sync_copy(st_v, out_hbm.at[wid])


def _sc_partials(gate_logits_flat):
  mesh = plsc.VectorSubcoreMesh(
      core_axis_name="c", subcore_axis_name="s",
      num_cores=_NC, num_subcores=_NS)
  kfn = pl.kernel(
      _sc_body,
      out_type=jax.ShapeDtypeStruct((_NW, 2 * _KV, _L), jnp.float32),
      mesh=mesh,
      scratch_types=[
          pltpu.VMEM((_TPW * _NUM_EXPERTS,), jnp.float32),
          pltpu.VMEM((2 * _KV, _L), jnp.float32),
      ],
      compiler_params=pltpu.CompilerParams(needs_layout_passes=False),
  )
  return kfn(gate_logits_flat)


def _finish_body(p_ref, o_ref):
  p = p_ref[...]
  cnt = jnp.sum(p[:, :_KV, :], axis=0)
  sw = jnp.sum(p[:, _KV:, :], axis=0)
  scale = _AUX_LOSS_COEF * _NUM_EXPERTS / float(_TOKENS) / float(_TOKENS)
  o_ref[...] = (jnp.sum(cnt * sw) * scale).reshape(1, 1)


def _finish(partials):
  return pl.pallas_call(
      _finish_body,
      out_shape=jax.ShapeDtypeStruct((1, 1), jnp.float32),
  )(partials)


@jax.jit
def kernel(gate_logits, attention_mask):
  del attention_mask
  partials = _sc_partials(gate_logits.reshape(-1))
  scale = _AUX_LOSS_COEF * _NUM_EXPERTS / float(_TOKENS) / float(_TOKENS)
  cnt = jnp.sum(partials[:, :_KV, :], axis=0)
  sw = jnp.sum(partials[:, _KV:, :], axis=0)
  return jnp.sum(cnt * sw) * scale


# speedup vs baseline: 1.2214x; 1.0169x over previous
import functools

import jax
import jax.numpy as jnp
from jax import lax
from jax.experimental import pallas as pl
from jax.experimental.pallas import tpu as pltpu
from jax.experimental.pallas import tpu_sc as plsc

_NUM_EXPERTS = 64
_TOKENS = 8192
_TOP_P = 0.8
_AUX_LOSS_COEF = 0.01

_NC = 2
_NS = 16
_L = 16
_NW = _NC * _NS
_TPW = _TOKENS // _NW
_KV = _NUM_EXPERTS // _L


def _merge16(a, b):
  rb = lax.rev(b, (0,))
  lo = jnp.minimum(a, rb)
  hi = jnp.maximum(a, rb)
  return lax.sort(lo), lax.sort(hi)


def _merge32(a0, a1, b0, b1):
  rb0 = lax.rev(b1, (0,))
  rb1 = lax.rev(b0, (0,))
  lo0 = jnp.minimum(a0, rb0)
  lo1 = jnp.minimum(a1, rb1)
  hi0 = jnp.maximum(a0, rb0)
  hi1 = jnp.maximum(a1, rb1)
  m0 = jnp.minimum(lo0, lo1)
  m1 = jnp.maximum(lo0, lo1)
  m2 = jnp.minimum(hi0, hi1)
  m3 = jnp.maximum(hi0, hi1)
  return lax.sort(m0), lax.sort(m1), lax.sort(m2), lax.sort(m3)


def _sort64(v):
  s = [lax.sort(x) for x in v]
  a0, a1 = _merge16(s[0], s[1])
  b0, b1 = _merge16(s[2], s[3])
  return _merge32(a0, a1, b0, b1)


def _token_step(lg_v, t, accs):
  off = t * _NUM_EXPERTS
  u = [jnp.exp(lg_v[pl.ds(off + k * _L, _L)]) for k in range(_KV)]
  s_tot = jnp.sum(u[0] + u[1] + u[2] + u[3])
  r = _sort64(u)
  t0 = jnp.sum(r[0])
  t1 = jnp.sum(r[1])
  t2 = jnp.sum(r[2])
  c0 = plsc.cumsum(r[0])
  c1 = plsc.cumsum(r[1]) + t0
  c2 = plsc.cumsum(r[2]) + (t0 + t1)
  c3 = plsc.cumsum(r[3]) + (t0 + t1 + t2)
  thr = (1.0 - _TOP_P) * s_tot
  big = jnp.full((_L,), 3.0e38, jnp.float32)
  cand = jnp.minimum(
      jnp.minimum(jnp.where(c0 >= thr, r[0], big),
                  jnp.where(c1 >= thr, r[1], big)),
      jnp.minimum(jnp.where(c2 >= thr, r[2], big),
                  jnp.where(c3 >= thr, r[3], big)))
  tau = jnp.min(cand)
  rinv = jnp.ones((_L,), jnp.float32) / s_tot
  new = []
  for k in range(_KV):
    cnt, sw = accs[2 * k], accs[2 * k + 1]
    new.append(cnt + jnp.where(u[k] >= tau, 1.0, 0.0))
    new.append(sw + u[k] * rinv)
  return tuple(new)


def _two_token_step(lg_v, i, accs):
  mid = _token_step(lg_v, 2 * i, accs)
  return _token_step(lg_v, 2 * i + 1, mid)


_NCHUNK = 4
_CTOK = _TPW // _NCHUNK


def _sc_body(lg_hbm, out_hbm, lg_v, st_v, sem0, sem1):
  wid = lax.axis_index("s") * _NC + lax.axis_index("c")
  base = wid * _TPW * _NUM_EXPERTS
  sems = (sem0, sem1)
  csz = _CTOK * _NUM_EXPERTS

  def copy(c):
    return pltpu.make_async_copy(
        lg_hbm.at[pl.ds(base + c * csz, csz)],
        lg_v.at[pl.ds((c % 2) * csz, csz)], sems[c % 2])

  copy(0).start()
  zero = jnp.zeros((_L,), jnp.float32)
  accs = tuple(zero for _ in range(2 * _KV))
  for c in range(_NCHUNK):
    if c + 1 < _NCHUNK:
      copy(c + 1).start()
    copy(c).wait()
    boff = (c % 2) * _CTOK

    def two_step(i, carry, boff=boff):
      mid = _token_step(lg_v, boff + 2 * i, carry)
      return _token_step(lg_v, boff + 2 * i + 1, mid)

    accs = lax.fori_loop(0, _CTOK // 2, two_step, accs)

  for k in range(_KV):
    st_v[k, :] = accs[2 * k]
    st_v[_KV + k, :] = accs[2 * k + 1]
  pltpu.sync_copy(st_v, out_hbm.at[wid])


def _sc_partials(gate_logits_flat):
  mesh = plsc.VectorSubcoreMesh(
      core_axis_name="c", subcore_axis_name="s",
      num_cores=_NC, num_subcores=_NS)
  kfn = pl.kernel(
      _sc_body,
      out_type=jax.ShapeDtypeStruct((_NW, 2 * _KV, _L), jnp.float32),
      mesh=mesh,
      scratch_types=[
          pltpu.VMEM((2 * _CTOK * _NUM_EXPERTS,), jnp.float32),
          pltpu.VMEM((2 * _KV, _L), jnp.float32),
          pltpu.SemaphoreType.DMA,
          pltpu.SemaphoreType.DMA,
      ],
      compiler_params=pltpu.CompilerParams(needs_layout_passes=False),
  )
  return kfn(gate_logits_flat)


def _finish_body(p_ref, o_ref):
  p = p_ref[...]
  cnt = jnp.sum(p[:, :_KV, :], axis=0)
  sw = jnp.sum(p[:, _KV:, :], axis=0)
  scale = _AUX_LOSS_COEF * _NUM_EXPERTS / float(_TOKENS) / float(_TOKENS)
  o_ref[...] = (jnp.sum(cnt * sw) * scale).reshape(1, 1)


def _finish(partials):
  return pl.pallas_call(
      _finish_body,
      out_shape=jax.ShapeDtypeStruct((1, 1), jnp.float32),
  )(partials)


@jax.jit
def kernel(gate_logits, attention_mask):
  del attention_mask
  partials = _sc_partials(gate_logits.reshape(-1))
  return _finish(partials)[0, 0]


# speedup vs baseline: 1.2626x; 1.0338x over previous
import functools

import jax
import jax.numpy as jnp
from jax import lax
from jax.experimental import pallas as pl
from jax.experimental.pallas import tpu as pltpu
from jax.experimental.pallas import tpu_sc as plsc

_NUM_EXPERTS = 64
_TOKENS = 8192
_TOP_P = 0.8
_AUX_LOSS_COEF = 0.01

_NC = 2
_NS = 16
_L = 16
_NW = _NC * _NS
_TPW = _TOKENS // _NW
_KV = _NUM_EXPERTS // _L


def _merge16(a, b):
  rb = lax.rev(b, (0,))
  lo = jnp.minimum(a, rb)
  hi = jnp.maximum(a, rb)
  return lax.sort(lo), lax.sort(hi)


def _merge32(a0, a1, b0, b1):
  rb0 = lax.rev(b1, (0,))
  rb1 = lax.rev(b0, (0,))
  lo0 = jnp.minimum(a0, rb0)
  lo1 = jnp.minimum(a1, rb1)
  hi0 = jnp.maximum(a0, rb0)
  hi1 = jnp.maximum(a1, rb1)
  m0 = jnp.minimum(lo0, lo1)
  m1 = jnp.maximum(lo0, lo1)
  m2 = jnp.minimum(hi0, hi1)
  m3 = jnp.maximum(hi0, hi1)
  return lax.sort(m0), lax.sort(m1), lax.sort(m2), lax.sort(m3)


def _sort64(v):
  s = [lax.sort(x) for x in v]
  a0, a1 = _merge16(s[0], s[1])
  b0, b1 = _merge16(s[2], s[3])
  return _merge32(a0, a1, b0, b1)


def _token_step(lg_v, t, accs):
  off = t * _NUM_EXPERTS
  u = [jnp.exp(lg_v[pl.ds(off + k * _L, _L)]) for k in range(_KV)]
  s_tot = jnp.sum(u[0] + u[1] + u[2] + u[3])
  r = _sort64(u)
  t0 = jnp.sum(r[0])
  t1 = jnp.sum(r[1])
  t2 = jnp.sum(r[2])
  c0 = plsc.cumsum(r[0])
  c1 = plsc.cumsum(r[1]) + t0
  c2 = plsc.cumsum(r[2]) + (t0 + t1)
  c3 = plsc.cumsum(r[3]) + (t0 + t1 + t2)
  thr = (1.0 - _TOP_P) * s_tot
  big = jnp.full((_L,), 3.0e38, jnp.float32)
  cand = jnp.minimum(
      jnp.minimum(jnp.where(c0 >= thr, r[0], big),
                  jnp.where(c1 >= thr, r[1], big)),
      jnp.minimum(jnp.where(c2 >= thr, r[2], big),
                  jnp.where(c3 >= thr, r[3], big)))
  tau = jnp.min(cand)
  rinv = jnp.ones((_L,), jnp.float32) / s_tot
  new = []
  for k in range(_KV):
    cnt, sw = accs[2 * k], accs[2 * k + 1]
    new.append(cnt + jnp.where(u[k] >= tau, 1.0, 0.0))
    new.append(sw + u[k] * rinv)
  return tuple(new)


def _two_token_step(lg_v, i, accs):
  mid = _token_step(lg_v, 2 * i, accs)
  return _token_step(lg_v, 2 * i + 1, mid)


def _sc_body(lg_hbm, out_hbm, lg_v, st_v):
  wid = lax.axis_index("s") * _NC + lax.axis_index("c")
  base = wid * _TPW * _NUM_EXPERTS
  pltpu.sync_copy(lg_hbm.at[pl.ds(base, _TPW * _NUM_EXPERTS)], lg_v)

  zero = jnp.zeros((_L,), jnp.float32)
  init = tuple(zero for _ in range(2 * _KV))
  accs = lax.fori_loop(0, _TPW // 2,
                       functools.partial(_two_token_step, lg_v), init)

  for k in range(_KV):
    st_v[k, :] = accs[2 * k]
    st_v[_KV + k, :] = accs[2 * k + 1]
  pltpu.sync_copy(st_v, out_hbm.at[wid])


def _sc_partials(gate_logits_flat):
  mesh = plsc.VectorSubcoreMesh(
      core_axis_name="c", subcore_axis_name="s",
      num_cores=_NC, num_subcores=_NS)
  kfn = pl.kernel(
      _sc_body,
      out_type=jax.ShapeDtypeStruct((_NW, 2 * _KV, _L), jnp.float32),
      mesh=mesh,
      scratch_types=[
          pltpu.VMEM((_TPW * _NUM_EXPERTS,), jnp.float32),
          pltpu.VMEM((2 * _KV, _L), jnp.float32),
      ],
      compiler_params=pltpu.CompilerParams(needs_layout_passes=False),
  )
  return kfn(gate_logits_flat)


def _finish_body(p_ref, o_ref):
  p = p_ref[...]
  cnt = jnp.sum(p[:, :_KV, :], axis=0)
  sw = jnp.sum(p[:, _KV:, :], axis=0)
  scale = _AUX_LOSS_COEF * _NUM_EXPERTS / float(_TOKENS) / float(_TOKENS)
  o_ref[...] = (jnp.sum(cnt * sw) * scale).reshape(1, 1)


def _finish(partials):
  return pl.pallas_call(
      _finish_body,
      out_shape=jax.ShapeDtypeStruct((1, 1), jnp.float32),
  )(partials)


@jax.jit
def kernel(gate_logits, attention_mask):
  del attention_mask
  partials = _sc_partials(gate_logits.reshape(-1))
  return _finish(partials)[0, 0]
